# baseline (speedup 1.0000x reference)
import jax
import jax.numpy as jnp
from jax import lax
from jax.experimental import pallas as pl
from jax.experimental.pallas import tpu as pltpu
from jax.experimental.pallas import tpu_sc as plsc

NC = 2
NS = 16
L = 16
NW = NC * NS
FG = 4
ES = NW // FG

_SC_PARAMS = pltpu.CompilerParams(
    needs_layout_passes=False, use_tc_tiling_on_sc=False)


def _build_sc_scalar_agg(e, n, np_, gather):
    ew_per_worker = e // NW
    assert e % NW == 0 and ew_per_worker % L == 0
    mesh = plsc.VectorSubcoreMesh(core_axis_name="c", subcore_axis_name="s")

    scratch = [
        pltpu.VMEM((ew_per_worker,), jnp.int32),
        pltpu.VMEM((ew_per_worker,), jnp.float32),
        pltpu.VMEM((np_,), jnp.float32),
        pltpu.SemaphoreType.DMA,
    ]
    if gather:
        scratch += [
            pltpu.VMEM((ew_per_worker,), jnp.int32),
            pltpu.VMEM((n,), jnp.float32),
        ]

    def body(*refs):
        if gather:
            (src_hbm, dst_hbm, ew_hbm, tab_hbm, out_hbm,
             didx, evals, acc, sem, sidx, tab) = refs
        else:
            (dst_hbm, ew_hbm, out_hbm, didx, evals, acc, sem) = refs
        cid = lax.axis_index("c")
        sid = lax.axis_index("s")
        wid = sid * NC + cid
        base = wid * ew_per_worker

        pltpu.async_copy(dst_hbm.at[pl.ds(base, ew_per_worker)], didx, sem)
        pltpu.async_copy(ew_hbm.at[pl.ds(base, ew_per_worker)], evals, sem)
        if gather:
            pltpu.async_copy(src_hbm.at[pl.ds(base, ew_per_worker)], sidx, sem)
            pltpu.async_copy(tab_hbm, tab, sem)

        @plsc.parallel_loop(0, np_ // L, unroll=8)
        def zero(i):
            acc[pl.ds(i * L, L)] = jnp.zeros((L,), jnp.float32)

        pltpu.make_async_copy(dst_hbm.at[pl.ds(0, ew_per_worker)], didx, sem).wait()
        pltpu.make_async_copy(ew_hbm.at[pl.ds(0, ew_per_worker)], evals, sem).wait()
        if gather:
            pltpu.make_async_copy(src_hbm.at[pl.ds(0, ew_per_worker)], sidx, sem).wait()
            pltpu.make_async_copy(tab_hbm, tab, sem).wait()

        @plsc.parallel_loop(0, ew_per_worker // L, unroll=4)
        def grp(g):
            dv = didx[pl.ds(g * L, L)]
            ev = evals[pl.ds(g * L, L)]
            if gather:
                sv = sidx[pl.ds(g * L, L)]
                ev = ev * plsc.load_gather(tab, [sv])
            plsc.addupdate_scatter(acc, [dv], ev)

        pltpu.sync_copy(acc, out_hbm.at[wid])

    return pl.kernel(
        body,
        out_type=jax.ShapeDtypeStruct((NW, np_), jnp.float32),
        mesh=mesh,
        scratch_types=scratch,
        compiler_params=_SC_PARAMS,
    )


def _build_sc_row_agg(e, n, np_, ch):
    rows = e // ch
    assert e % ch == 0 and rows % ES == 0 and ch % L == 0
    cpt = rows // ES
    mesh = plsc.VectorSubcoreMesh(core_axis_name="c", subcore_axis_name="s")

    scratch = [
        pltpu.VMEM((FG * n,), jnp.float32),
        pltpu.VMEM((FG * np_,), jnp.float32),
        pltpu.VMEM((ch,), jnp.int32),
        pltpu.VMEM((ch,), jnp.int32),
        pltpu.VMEM((ch,), jnp.int32),
        pltpu.VMEM((ch,), jnp.int32),
        pltpu.VMEM((ch,), jnp.float32),
        pltpu.VMEM((ch,), jnp.float32),
        pltpu.SemaphoreType.DMA,
        pltpu.SemaphoreType.DMA,
        pltpu.SemaphoreType.DMA,
    ]

    def body(src2d, dst2d, ew2d, hpt4, out_hbm,
             hq, acc, sb0, sb1, db0, db1, eb0, eb1, sem0, sem1, semh):
        cid = lax.axis_index("c")
        sid = lax.axis_index("s")
        wid = sid * NC + cid
        fg = wid % FG
        es = wid // FG
        sb = (sb0, sb1)
        db = (db0, db1)
        eb = (eb0, eb1)
        sems = (sem0, sem1)

        def start(b, ci):
            row = es * cpt + ci
            pltpu.async_copy(src2d.at[row], sb[b], sems[b])
            pltpu.async_copy(dst2d.at[row], db[b], sems[b])
            pltpu.async_copy(ew2d.at[row], eb[b], sems[b])

        def drain(b):
            pltpu.make_async_copy(src2d.at[0], sb[b], sems[b]).wait()
            pltpu.make_async_copy(dst2d.at[0], db[b], sems[b]).wait()
            pltpu.make_async_copy(ew2d.at[0], eb[b], sems[b]).wait()

        start(0, 0)
        pltpu.async_copy(hpt4.at[fg], hq, semh)

        @plsc.parallel_loop(0, (FG * np_) // L, unroll=8)
        def zero(i):
            acc[pl.ds(i * L, L)] = jnp.zeros((L,), jnp.float32)

        pltpu.make_async_copy(hpt4.at[0], hq, semh).wait()

        coff_n = [jnp.full((L,), c * n, jnp.int32) for c in range(FG)]
        coff_p = [jnp.full((L,), c * np_, jnp.int32) for c in range(FG)]

        def outer(k, carry):
            for b in range(2):
                ci = k * 2 + b
                drain(b)

                @pl.when(ci + 1 < cpt)
                def _():
                    start(1 - b, ci + 1)

                @plsc.parallel_loop(0, ch // L, unroll=8)
                def grp(g):
                    sv = sb[b][pl.ds(g * L, L)]
                    dv = db[b][pl.ds(g * L, L)]
                    ev = eb[b][pl.ds(g * L, L)]
                    for c in range(FG):
                        gv = plsc.load_gather(hq, [sv + coff_n[c]])
                        plsc.addupdate_scatter(acc, [dv + coff_p[c]], gv * ev)
            return carry
        lax.fori_loop(0, cpt // 2, outer, 0)

        pltpu.sync_copy(acc, out_hbm.at[wid])

    return pl.kernel(
        body,
        out_type=jax.ShapeDtypeStruct((NW, FG * np_), jnp.float32),
        mesh=mesh,
        scratch_types=scratch,
        compiler_params=_SC_PARAMS,
    )


def _tc_call(body, out_shapes):
    return pl.pallas_call(body, out_shape=out_shapes)


def _tc_norm(degp, w1, x, n):
    def body(degp_ref, w_ref, x_ref, dinv_ref, hp_ref):
        deg = jnp.sum(degp_ref[...], axis=0) + 1.0
        dinv = lax.rsqrt(deg)[:n][None, :]
        dinv_ref[...] = dinv
        hmat_t = lax.dot_general(
            w_ref[...], x_ref[...], (((0,), (1,)), ((), ())),
            preferred_element_type=jnp.float32)
        hp_ref[...] = hmat_t * dinv
    hdim = w1.shape[1]
    return _tc_call(body, (
        jax.ShapeDtypeStruct((1, n), jnp.float32),
        jax.ShapeDtypeStruct((hdim, n), jnp.float32),
    ))(degp, w1, x)


def _tc_layer1_combine(aggp, hp_t, dinv_t, w2, b1c, n, np_, h):
    def body(aggp_ref, hp_ref, dinv_ref, w2_ref, b1_ref, gp_ref):
        a = aggp_ref[...].reshape(ES, FG, FG, np_)
        agg_t = jnp.sum(a, axis=0).reshape(h, np_)[:, :n]
        dv = dinv_ref[...]
        a1 = jnp.maximum(dv * (agg_t + hp_ref[...]) + b1_ref[...], 0.0)
        g = lax.dot_general(w2_ref[...], a1, (((0,), (0,)), ((), ())),
                            preferred_element_type=jnp.float32)
        gp_ref[...] = dv * g
    return _tc_call(body, jax.ShapeDtypeStruct((1, n), jnp.float32))(
        aggp, hp_t, dinv_t, w2, b1c)


def _tc_layer2_combine(agg2p, gp_t, dinv_t, b2c, n):
    def body(a2_ref, gp_ref, dinv_ref, b2_ref, o_ref):
        agg2 = jnp.sum(a2_ref[...], axis=0)[:n][None, :]
        o_ref[...] = dinv_ref[...] * (agg2 + gp_ref[...]) + b2_ref[...]
    return _tc_call(body, jax.ShapeDtypeStruct((1, n), jnp.float32))(
        agg2p, gp_t, dinv_t, b2c)


def kernel(x, edge_index, edge_weight, W1, b1, W2, b2):
    n, d = x.shape
    h = W1.shape[1]
    e = edge_index.shape[1]
    assert h == FG * FG
    np_ = ((n + 127) // 128) * 128

    src = edge_index[0].astype(jnp.int32)
    dst = edge_index[1].astype(jnp.int32)
    ew = edge_weight.astype(jnp.float32)

    ch = 4000
    while e % ch != 0 or (e // ch) % ES != 0 or ((e // ch) // ES) % 2 != 0:
        ch -= 8

    degp = _build_sc_scalar_agg(e, n, np_, gather=False)(dst, ew)

    dinv_t, hp_t = _tc_norm(degp, W1, x, n)

    aggp = _build_sc_row_agg(e, n, np_, ch)(
        src.reshape(e // ch, ch), dst.reshape(e // ch, ch),
        ew.reshape(e // ch, ch), hp_t.reshape(FG, FG * n))

    gp_t = _tc_layer1_combine(
        aggp.reshape(NW, FG, np_), hp_t, dinv_t, W2, b1.reshape(h, 1),
        n, np_, h)

    agg2p = _build_sc_scalar_agg(e, n, np_, gather=True)(
        src, dst, ew, gp_t.reshape(n))

    out = _tc_layer2_combine(agg2p, gp_t, dinv_t, b2.reshape(1, 1), n)
    return out[0]


# speedup vs baseline: 1.0105x; 1.0105x over previous
import jax
import jax.numpy as jnp
from jax import lax
from jax.experimental import pallas as pl
from jax.experimental.pallas import tpu as pltpu
from jax.experimental.pallas import tpu_sc as plsc

NC = 2
NS = 16
L = 16
NW = NC * NS
FG = 4
ES = NW // FG

_SC_PARAMS = pltpu.CompilerParams(
    needs_layout_passes=False, use_tc_tiling_on_sc=False)


def _build_sc_scalar_agg(e, n, np_, gather):
    ew_per_worker = e // NW
    assert e % NW == 0 and ew_per_worker % L == 0
    mesh = plsc.VectorSubcoreMesh(core_axis_name="c", subcore_axis_name="s")

    scratch = [
        pltpu.VMEM((ew_per_worker,), jnp.int32),
        pltpu.VMEM((ew_per_worker,), jnp.float32),
        pltpu.VMEM((np_,), jnp.float32),
        pltpu.SemaphoreType.DMA,
    ]
    if gather:
        scratch += [
            pltpu.VMEM((ew_per_worker,), jnp.int32),
            pltpu.VMEM((n,), jnp.float32),
        ]

    def body(*refs):
        if gather:
            (src_hbm, dst_hbm, ew_hbm, tab_hbm, out_hbm,
             didx, evals, acc, sem, sidx, tab) = refs
        else:
            (dst_hbm, ew_hbm, out_hbm, didx, evals, acc, sem) = refs
        cid = lax.axis_index("c")
        sid = lax.axis_index("s")
        wid = sid * NC + cid
        base = wid * ew_per_worker

        pltpu.async_copy(dst_hbm.at[pl.ds(base, ew_per_worker)], didx, sem)
        pltpu.async_copy(ew_hbm.at[pl.ds(base, ew_per_worker)], evals, sem)
        if gather:
            pltpu.async_copy(src_hbm.at[pl.ds(base, ew_per_worker)], sidx, sem)
            pltpu.async_copy(tab_hbm, tab, sem)

        @plsc.parallel_loop(0, np_ // L, unroll=8)
        def zero(i):
            acc[pl.ds(i * L, L)] = jnp.zeros((L,), jnp.float32)

        pltpu.make_async_copy(dst_hbm.at[pl.ds(0, ew_per_worker)], didx, sem).wait()
        pltpu.make_async_copy(ew_hbm.at[pl.ds(0, ew_per_worker)], evals, sem).wait()
        if gather:
            pltpu.make_async_copy(src_hbm.at[pl.ds(0, ew_per_worker)], sidx, sem).wait()
            pltpu.make_async_copy(tab_hbm, tab, sem).wait()

        @plsc.parallel_loop(0, ew_per_worker // L, unroll=4)
        def grp(g):
            dv = didx[pl.ds(g * L, L)]
            ev = evals[pl.ds(g * L, L)]
            if gather:
                sv = sidx[pl.ds(g * L, L)]
                ev = ev * plsc.load_gather(tab, [sv])
            plsc.addupdate_scatter(acc, [dv], ev)

        pltpu.sync_copy(acc, out_hbm.at[wid])

    return pl.kernel(
        body,
        out_type=jax.ShapeDtypeStruct((NW, np_), jnp.float32),
        mesh=mesh,
        scratch_types=scratch,
        compiler_params=_SC_PARAMS,
    )


def _build_sc_row_agg(e, n, np_, ch):
    rows = e // ch
    assert e % ch == 0 and rows % ES == 0 and ch % L == 0
    cpt = rows // ES
    mesh = plsc.VectorSubcoreMesh(core_axis_name="c", subcore_axis_name="s")

    scratch = [
        pltpu.VMEM((FG * n,), jnp.float32),
        pltpu.VMEM((FG * np_,), jnp.float32),
        pltpu.VMEM((ch,), jnp.int32),
        pltpu.VMEM((ch,), jnp.int32),
        pltpu.VMEM((ch,), jnp.int32),
        pltpu.VMEM((ch,), jnp.int32),
        pltpu.VMEM((ch,), jnp.float32),
        pltpu.VMEM((ch,), jnp.float32),
        pltpu.SemaphoreType.DMA,
        pltpu.SemaphoreType.DMA,
        pltpu.SemaphoreType.DMA,
    ]

    def body(src2d, dst2d, ew2d, hpt4, out_hbm,
             hq, acc, sb0, sb1, db0, db1, eb0, eb1, sem0, sem1, semh):
        cid = lax.axis_index("c")
        sid = lax.axis_index("s")
        wid = sid * NC + cid
        fg = wid % FG
        es = wid // FG
        sb = (sb0, sb1)
        db = (db0, db1)
        eb = (eb0, eb1)
        sems = (sem0, sem1)

        def start(b, ci):
            row = es * cpt + ci
            pltpu.async_copy(src2d.at[row], sb[b], sems[b])
            pltpu.async_copy(dst2d.at[row], db[b], sems[b])
            pltpu.async_copy(ew2d.at[row], eb[b], sems[b])

        def drain(b):
            pltpu.make_async_copy(src2d.at[0], sb[b], sems[b]).wait()
            pltpu.make_async_copy(dst2d.at[0], db[b], sems[b]).wait()
            pltpu.make_async_copy(ew2d.at[0], eb[b], sems[b]).wait()

        start(0, 0)
        pltpu.async_copy(hpt4.at[fg], hq, semh)

        @plsc.parallel_loop(0, (FG * np_) // L, unroll=8)
        def zero(i):
            acc[pl.ds(i * L, L)] = jnp.zeros((L,), jnp.float32)

        pltpu.make_async_copy(hpt4.at[0], hq, semh).wait()

        coff_n = [jnp.full((L,), c * n, jnp.int32) for c in range(FG)]
        coff_p = [jnp.full((L,), c * np_, jnp.int32) for c in range(FG)]

        def outer(k, carry):
            for b in range(2):
                ci = k * 2 + b
                drain(b)

                @pl.when(ci + 1 < cpt)
                def _():
                    start(1 - b, ci + 1)

                @plsc.parallel_loop(0, ch // L, unroll=4)
                def grp(g):
                    sv = sb[b][pl.ds(g * L, L)]
                    dv = db[b][pl.ds(g * L, L)]
                    ev = eb[b][pl.ds(g * L, L)]
                    for c in range(FG):
                        gv = plsc.load_gather(hq, [sv + coff_n[c]])
                        plsc.addupdate_scatter(acc, [dv + coff_p[c]], gv * ev)
            return carry
        lax.fori_loop(0, cpt // 2, outer, 0)

        pltpu.sync_copy(acc, out_hbm.at[wid])

    return pl.kernel(
        body,
        out_type=jax.ShapeDtypeStruct((NW, FG * np_), jnp.float32),
        mesh=mesh,
        scratch_types=scratch,
        compiler_params=_SC_PARAMS,
    )


def _tc_call(body, out_shapes):
    return pl.pallas_call(body, out_shape=out_shapes)


def _tc_norm(degp, w1, x, n):
    def body(degp_ref, w_ref, x_ref, dinv_ref, hp_ref):
        deg = jnp.sum(degp_ref[...], axis=0) + 1.0
        dinv = lax.rsqrt(deg)[:n][None, :]
        dinv_ref[...] = dinv
        hmat_t = lax.dot_general(
            w_ref[...], x_ref[...], (((0,), (1,)), ((), ())),
            preferred_element_type=jnp.float32)
        hp_ref[...] = hmat_t * dinv
    hdim = w1.shape[1]
    return _tc_call(body, (
        jax.ShapeDtypeStruct((1, n), jnp.float32),
        jax.ShapeDtypeStruct((hdim, n), jnp.float32),
    ))(degp, w1, x)


def _tc_layer1_combine(aggp, hp_t, dinv_t, w2, b1c, n, np_, h):
    def body(aggp_ref, hp_ref, dinv_ref, w2_ref, b1_ref, gp_ref):
        a = aggp_ref[...].reshape(ES, FG, FG, np_)
        agg_t = jnp.sum(a, axis=0).reshape(h, np_)[:, :n]
        dv = dinv_ref[...]
        a1 = jnp.maximum(dv * (agg_t + hp_ref[...]) + b1_ref[...], 0.0)
        g = lax.dot_general(w2_ref[...], a1, (((0,), (0,)), ((), ())),
                            preferred_element_type=jnp.float32)
        gp_ref[...] = dv * g
    return _tc_call(body, jax.ShapeDtypeStruct((1, n), jnp.float32))(
        aggp, hp_t, dinv_t, w2, b1c)


def _tc_layer2_combine(agg2p, gp_t, dinv_t, b2c, n):
    def body(a2_ref, gp_ref, dinv_ref, b2_ref, o_ref):
        agg2 = jnp.sum(a2_ref[...], axis=0)[:n][None, :]
        o_ref[...] = dinv_ref[...] * (agg2 + gp_ref[...]) + b2_ref[...]
    return _tc_call(body, jax.ShapeDtypeStruct((1, n), jnp.float32))(
        agg2p, gp_t, dinv_t, b2c)


def kernel(x, edge_index, edge_weight, W1, b1, W2, b2):
    n, d = x.shape
    h = W1.shape[1]
    e = edge_index.shape[1]
    assert h == FG * FG
    np_ = ((n + 127) // 128) * 128

    src = edge_index[0].astype(jnp.int32)
    dst = edge_index[1].astype(jnp.int32)
    ew = edge_weight.astype(jnp.float32)

    ch = 4000
    while e % ch != 0 or (e // ch) % ES != 0 or ((e // ch) // ES) % 2 != 0:
        ch -= 8

    degp = _build_sc_scalar_agg(e, n, np_, gather=False)(dst, ew)

    dinv_t, hp_t = _tc_norm(degp, W1, x, n)

    aggp = _build_sc_row_agg(e, n, np_, ch)(
        src.reshape(e // ch, ch), dst.reshape(e // ch, ch),
        ew.reshape(e // ch, ch), hp_t.reshape(FG, FG * n))

    gp_t = _tc_layer1_combine(
        aggp.reshape(NW, FG, np_), hp_t, dinv_t, W2, b1.reshape(h, 1),
        n, np_, h)

    agg2p = _build_sc_scalar_agg(e, n, np_, gather=True)(
        src, dst, ew, gp_t.reshape(n))

    out = _tc_layer2_combine(agg2p, gp_t, dinv_t, b2.reshape(1, 1), n)
    return out[0]


# speedup vs baseline: 1.0197x; 1.0091x over previous
import jax
import jax.numpy as jnp
from jax import lax
from jax.experimental import pallas as pl
from jax.experimental.pallas import tpu as pltpu
from jax.experimental.pallas import tpu_sc as plsc

NC = 2
NS = 16
L = 16
NW = NC * NS
FG = 4
ES = NW // FG

_SC_PARAMS = pltpu.CompilerParams(
    needs_layout_passes=False, use_tc_tiling_on_sc=False)


def _build_sc_scalar_agg(e, n, np_, gather):
    ew_per_worker = e // NW
    assert e % NW == 0 and ew_per_worker % L == 0
    mesh = plsc.VectorSubcoreMesh(core_axis_name="c", subcore_axis_name="s")

    scratch = [
        pltpu.VMEM((ew_per_worker,), jnp.int32),
        pltpu.VMEM((ew_per_worker,), jnp.float32),
        pltpu.VMEM((np_,), jnp.float32),
        pltpu.SemaphoreType.DMA,
    ]
    if gather:
        scratch += [
            pltpu.VMEM((ew_per_worker,), jnp.int32),
            pltpu.VMEM((n,), jnp.float32),
        ]

    def body(*refs):
        if gather:
            (src_hbm, dst_hbm, ew_hbm, tab_hbm, out_hbm,
             didx, evals, acc, sem, sidx, tab) = refs
        else:
            (dst_hbm, ew_hbm, out_hbm, didx, evals, acc, sem) = refs
        cid = lax.axis_index("c")
        sid = lax.axis_index("s")
        wid = sid * NC + cid
        base = wid * ew_per_worker

        pltpu.async_copy(dst_hbm.at[pl.ds(base, ew_per_worker)], didx, sem)
        pltpu.async_copy(ew_hbm.at[pl.ds(base, ew_per_worker)], evals, sem)
        if gather:
            pltpu.async_copy(src_hbm.at[pl.ds(base, ew_per_worker)], sidx, sem)
            pltpu.async_copy(tab_hbm.at[0], tab, sem)

        @plsc.parallel_loop(0, np_ // L, unroll=8)
        def zero(i):
            acc[pl.ds(i * L, L)] = jnp.zeros((L,), jnp.float32)

        pltpu.make_async_copy(dst_hbm.at[pl.ds(0, ew_per_worker)], didx, sem).wait()
        pltpu.make_async_copy(ew_hbm.at[pl.ds(0, ew_per_worker)], evals, sem).wait()
        if gather:
            pltpu.make_async_copy(src_hbm.at[pl.ds(0, ew_per_worker)], sidx, sem).wait()
            pltpu.make_async_copy(tab_hbm.at[0], tab, sem).wait()

        @plsc.parallel_loop(0, ew_per_worker // L, unroll=4)
        def grp(g):
            dv = didx[pl.ds(g * L, L)]
            ev = evals[pl.ds(g * L, L)]
            if gather:
                sv = sidx[pl.ds(g * L, L)]
                ev = ev * plsc.load_gather(tab, [sv])
            plsc.addupdate_scatter(acc, [dv], ev)

        pltpu.sync_copy(acc, out_hbm.at[wid])

    return pl.kernel(
        body,
        out_type=jax.ShapeDtypeStruct((NW, np_), jnp.float32),
        mesh=mesh,
        scratch_types=scratch,
        compiler_params=_SC_PARAMS,
    )


def _build_sc_row_agg(e, n, np_, ch, h):
    rows = e // ch
    assert e % ch == 0 and rows % ES == 0 and (rows // ES) % 2 == 0
    cpt = rows // ES
    mesh = plsc.VectorSubcoreMesh(core_axis_name="c", subcore_axis_name="s")

    scratch = [
        pltpu.VMEM((FG * n,), jnp.float32),
        pltpu.VMEM((FG, np_), jnp.float32),
        pltpu.VMEM((ch,), jnp.int32),
        pltpu.VMEM((ch,), jnp.int32),
        pltpu.VMEM((ch,), jnp.int32),
        pltpu.VMEM((ch,), jnp.int32),
        pltpu.VMEM((ch,), jnp.float32),
        pltpu.VMEM((ch,), jnp.float32),
        pltpu.SemaphoreType.DMA,
        pltpu.SemaphoreType.DMA,
        pltpu.SemaphoreType.DMA,
    ]

    def body(src_hbm, dst_hbm, ew_hbm, hpt, out_hbm,
             hq, acc, sb0, sb1, db0, db1, eb0, eb1, sem0, sem1, semh):
        cid = lax.axis_index("c")
        sid = lax.axis_index("s")
        wid = sid * NC + cid
        fg = wid % FG
        es = wid // FG
        sb = (sb0, sb1)
        db = (db0, db1)
        eb = (eb0, eb1)
        sems = (sem0, sem1)

        def start(b, ci):
            off = (es * cpt + ci) * ch
            pltpu.async_copy(src_hbm.at[pl.ds(off, ch)], sb[b], sems[b])
            pltpu.async_copy(dst_hbm.at[pl.ds(off, ch)], db[b], sems[b])
            pltpu.async_copy(ew_hbm.at[pl.ds(off, ch)], eb[b], sems[b])

        def drain(b):
            pltpu.make_async_copy(src_hbm.at[pl.ds(0, ch)], sb[b], sems[b]).wait()
            pltpu.make_async_copy(dst_hbm.at[pl.ds(0, ch)], db[b], sems[b]).wait()
            pltpu.make_async_copy(ew_hbm.at[pl.ds(0, ch)], eb[b], sems[b]).wait()

        start(0, 0)
        for c in range(FG):
            pltpu.async_copy(hpt.at[fg * FG + c], hq.at[pl.ds(c * n, n)], semh)

        @plsc.parallel_loop(0, np_ // L, unroll=4)
        def zero(i):
            for c in range(FG):
                acc[c, pl.ds(i * L, L)] = jnp.zeros((L,), jnp.float32)

        for c in range(FG):
            pltpu.make_async_copy(hpt.at[0], hq.at[pl.ds(0, n)], semh).wait()

        coff_n = [jnp.full((L,), c * n, jnp.int32) for c in range(FG)]
        crow = [jnp.full((L,), c, jnp.int32) for c in range(FG)]

        def outer(k, carry):
            for b in range(2):
                ci = k * 2 + b
                drain(b)

                @pl.when(ci + 1 < cpt)
                def _():
                    start(1 - b, ci + 1)

                @plsc.parallel_loop(0, ch // L, unroll=4)
                def grp(g):
                    sv = sb[b][pl.ds(g * L, L)]
                    dv = db[b][pl.ds(g * L, L)]
                    ev = eb[b][pl.ds(g * L, L)]
                    for c in range(FG):
                        gv = plsc.load_gather(hq, [sv + coff_n[c]])
                        plsc.addupdate_scatter(acc, [crow[c], dv], gv * ev)
            return carry
        lax.fori_loop(0, cpt // 2, outer, 0)

        pltpu.sync_copy(acc, out_hbm.at[pl.ds(wid * FG, FG)])

    return pl.kernel(
        body,
        out_type=jax.ShapeDtypeStruct((NW * FG, np_), jnp.float32),
        mesh=mesh,
        scratch_types=scratch,
        compiler_params=_SC_PARAMS,
    )


def _tc_call(body, out_shapes):
    return pl.pallas_call(body, out_shape=out_shapes)


def _tc_norm(degp, w1, x, n):
    def body(degp_ref, w_ref, x_ref, dinv_ref, hp_ref):
        deg = jnp.sum(degp_ref[...], axis=0) + 1.0
        dinv = lax.rsqrt(deg)[:n][None, :]
        dinv_ref[...] = dinv
        hmat_t = lax.dot_general(
            w_ref[...], x_ref[...], (((0,), (1,)), ((), ())),
            preferred_element_type=jnp.float32)
        hp_ref[...] = hmat_t * dinv
    hdim = w1.shape[1]
    return _tc_call(body, (
        jax.ShapeDtypeStruct((1, n), jnp.float32),
        jax.ShapeDtypeStruct((hdim, n), jnp.float32),
    ))(degp, w1, x)


def _tc_layer1_combine(aggp, hp_t, dinv_t, w2, b1c, n, np_, h):
    def body(aggp_ref, hp_ref, dinv_ref, w2_ref, b1_ref, gp_ref):
        a = aggp_ref[...].reshape(ES, h, np_)
        agg_t = jnp.sum(a, axis=0)[:, :n]
        dv = dinv_ref[...]
        a1 = jnp.maximum(dv * (agg_t + hp_ref[...]) + b1_ref[...], 0.0)
        g = lax.dot_general(w2_ref[...], a1, (((0,), (0,)), ((), ())),
                            preferred_element_type=jnp.float32)
        gp_ref[...] = dv * g
    return _tc_call(body, jax.ShapeDtypeStruct((1, n), jnp.float32))(
        aggp, hp_t, dinv_t, w2, b1c)


def _tc_layer2_combine(agg2p, gp_t, dinv_t, b2c, n):
    def body(a2_ref, gp_ref, dinv_ref, b2_ref, o_ref):
        agg2 = jnp.sum(a2_ref[...], axis=0)[:n][None, :]
        o_ref[...] = dinv_ref[...] * (agg2 + gp_ref[...]) + b2_ref[...]
    return _tc_call(body, jax.ShapeDtypeStruct((1, n), jnp.float32))(
        agg2p, gp_t, dinv_t, b2c)


def kernel(x, edge_index, edge_weight, W1, b1, W2, b2):
    n, d = x.shape
    h = W1.shape[1]
    e = edge_index.shape[1]
    assert h == FG * FG
    np_ = ((n + 127) // 128) * 128

    src = edge_index[0].astype(jnp.int32)
    dst = edge_index[1].astype(jnp.int32)
    ew = edge_weight.astype(jnp.float32)

    ch = 4000
    while e % ch != 0 or (e // ch) % ES != 0 or ((e // ch) // ES) % 2 != 0:
        ch -= 8

    degp = _build_sc_scalar_agg(e, n, np_, gather=False)(dst, ew)

    dinv_t, hp_t = _tc_norm(degp, W1, x, n)

    aggp = _build_sc_row_agg(e, n, np_, ch, h)(src, dst, ew, hp_t)

    gp_t = _tc_layer1_combine(
        aggp, hp_t, dinv_t, W2, b1.reshape(h, 1), n, np_, h)

    agg2p = _build_sc_scalar_agg(e, n, np_, gather=True)(
        src, dst, ew, gp_t)

    out = _tc_layer2_combine(agg2p, gp_t, dinv_t, b2.reshape(1, 1), n)
    return out[0]


# speedup vs baseline: 1.4010x; 1.3740x over previous
import jax
import jax.numpy as jnp
from jax import lax
from jax.experimental import pallas as pl
from jax.experimental.pallas import tpu as pltpu
from jax.experimental.pallas import tpu_sc as plsc

NC = 2
NS = 16
L = 16
NW = NC * NS
FG = 4
ES = NW // FG

_SC_PARAMS = pltpu.CompilerParams(
    needs_layout_passes=False, use_tc_tiling_on_sc=False)


def _build_sc_scalar_agg(e, n, np_, gather):
    ew_per_worker = e // NW
    assert e % NW == 0 and ew_per_worker % L == 0
    mesh = plsc.VectorSubcoreMesh(core_axis_name="c", subcore_axis_name="s")

    scratch = [
        pltpu.VMEM((ew_per_worker,), jnp.int32),
        pltpu.VMEM((ew_per_worker,), jnp.float32),
        pltpu.VMEM((np_,), jnp.float32),
        pltpu.SemaphoreType.DMA,
    ]
    if gather:
        scratch += [
            pltpu.VMEM((ew_per_worker,), jnp.int32),
            pltpu.VMEM((n,), jnp.float32),
        ]

    def body(*refs):
        if gather:
            (src_hbm, dst_hbm, ew_hbm, tab_hbm, out_hbm,
             didx, evals, acc, sem, sidx, tab) = refs
        else:
            (dst_hbm, ew_hbm, out_hbm, didx, evals, acc, sem) = refs
        cid = lax.axis_index("c")
        sid = lax.axis_index("s")
        wid = sid * NC + cid
        base = wid * ew_per_worker

        pltpu.async_copy(dst_hbm.at[pl.ds(base, ew_per_worker)], didx, sem)
        pltpu.async_copy(ew_hbm.at[pl.ds(base, ew_per_worker)], evals, sem)
        if gather:
            pltpu.async_copy(src_hbm.at[pl.ds(base, ew_per_worker)], sidx, sem)
            pltpu.async_copy(tab_hbm, tab, sem)

        @plsc.parallel_loop(0, np_ // L, unroll=8)
        def zero(i):
            acc[pl.ds(i * L, L)] = jnp.zeros((L,), jnp.float32)

        pltpu.make_async_copy(dst_hbm.at[pl.ds(0, ew_per_worker)], didx, sem).wait()
        pltpu.make_async_copy(ew_hbm.at[pl.ds(0, ew_per_worker)], evals, sem).wait()
        if gather:
            pltpu.make_async_copy(src_hbm.at[pl.ds(0, ew_per_worker)], sidx, sem).wait()
            pltpu.make_async_copy(tab_hbm, tab, sem).wait()

        @plsc.parallel_loop(0, ew_per_worker // L, unroll=4)
        def grp(g):
            dv = didx[pl.ds(g * L, L)]
            ev = evals[pl.ds(g * L, L)]
            if gather:
                sv = sidx[pl.ds(g * L, L)]
                ev = ev * plsc.load_gather(tab, [sv])
            plsc.addupdate_scatter(acc, [dv], ev)

        pltpu.sync_copy(acc, out_hbm.at[pl.ds(wid * np_, np_)])

    return pl.kernel(
        body,
        out_type=jax.ShapeDtypeStruct((NW * np_,), jnp.float32),
        mesh=mesh,
        scratch_types=scratch,
        compiler_params=_SC_PARAMS,
    )


def _build_sc_row_agg(e, n, np_, ch, h):
    rows = e // ch
    assert e % ch == 0 and rows % ES == 0 and (rows // ES) % 2 == 0
    cpt = rows // ES
    mesh = plsc.VectorSubcoreMesh(core_axis_name="c", subcore_axis_name="s")

    scratch = [
        pltpu.VMEM((FG * n,), jnp.float32),
        pltpu.VMEM((FG * np_,), jnp.float32),
        pltpu.VMEM((ch,), jnp.int32),
        pltpu.VMEM((ch,), jnp.int32),
        pltpu.VMEM((ch,), jnp.int32),
        pltpu.VMEM((ch,), jnp.int32),
        pltpu.VMEM((ch,), jnp.float32),
        pltpu.VMEM((ch,), jnp.float32),
        pltpu.SemaphoreType.DMA,
        pltpu.SemaphoreType.DMA,
        pltpu.SemaphoreType.DMA,
    ]

    def body(src_hbm, dst_hbm, ew_hbm, hpt, out_hbm,
             hq, acc, sb0, sb1, db0, db1, eb0, eb1, sem0, sem1, semh):
        cid = lax.axis_index("c")
        sid = lax.axis_index("s")
        wid = sid * NC + cid
        fg = wid % FG
        es = wid // FG
        sb = (sb0, sb1)
        db = (db0, db1)
        eb = (eb0, eb1)
        sems = (sem0, sem1)

        def start(b, ci):
            off = (es * cpt + ci) * ch
            pltpu.async_copy(src_hbm.at[pl.ds(off, ch)], sb[b], sems[b])
            pltpu.async_copy(dst_hbm.at[pl.ds(off, ch)], db[b], sems[b])
            pltpu.async_copy(ew_hbm.at[pl.ds(off, ch)], eb[b], sems[b])

        def drain(b):
            pltpu.make_async_copy(src_hbm.at[pl.ds(0, ch)], sb[b], sems[b]).wait()
            pltpu.make_async_copy(dst_hbm.at[pl.ds(0, ch)], db[b], sems[b]).wait()
            pltpu.make_async_copy(ew_hbm.at[pl.ds(0, ch)], eb[b], sems[b]).wait()

        start(0, 0)
        pltpu.async_copy(hpt.at[pl.ds(fg * FG * n, FG * n)], hq, semh)

        @plsc.parallel_loop(0, (FG * np_) // L, unroll=8)
        def zero(i):
            acc[pl.ds(i * L, L)] = jnp.zeros((L,), jnp.float32)

        pltpu.make_async_copy(hpt.at[pl.ds(0, FG * n)], hq, semh).wait()

        coff_n = [jnp.full((L,), c * n, jnp.int32) for c in range(FG)]
        coff_p = [jnp.full((L,), c * np_, jnp.int32) for c in range(FG)]

        def outer(k, carry):
            for b in range(2):
                ci = k * 2 + b
                drain(b)

                @pl.when(ci + 1 < cpt)
                def _():
                    start(1 - b, ci + 1)

                @plsc.parallel_loop(0, ch // L, unroll=4)
                def grp(g):
                    sv = sb[b][pl.ds(g * L, L)]
                    dv = db[b][pl.ds(g * L, L)]
                    ev = eb[b][pl.ds(g * L, L)]
                    for c in range(FG):
                        gv = plsc.load_gather(hq, [sv + coff_n[c]])
                        plsc.addupdate_scatter(acc, [dv + coff_p[c]], gv * ev)
            return carry
        lax.fori_loop(0, cpt // 2, outer, 0)

        pltpu.sync_copy(acc, out_hbm.at[pl.ds(wid * FG * np_, FG * np_)])

    return pl.kernel(
        body,
        out_type=jax.ShapeDtypeStruct((NW * FG * np_,), jnp.float32),
        mesh=mesh,
        scratch_types=scratch,
        compiler_params=_SC_PARAMS,
    )


def _tc_call(body, out_shapes):
    return pl.pallas_call(body, out_shape=out_shapes)


def _tc_split_edges(edge_index, e):
    def body(ei_ref, s_ref, d_ref):
        v = ei_ref[...]
        s_ref[...] = v[0]
        d_ref[...] = v[1]
    return _tc_call(body, (
        jax.ShapeDtypeStruct((e,), jnp.int32),
        jax.ShapeDtypeStruct((e,), jnp.int32),
    ))(edge_index)


def _tc_norm(degp, w1, x, n, np_, h):
    def body(degp_ref, w_ref, x_ref, dinv_ref, hp_ref):
        dp = degp_ref[...]
        deg = jnp.full((n,), 1.0, jnp.float32)
        for w in range(NW):
            deg = deg + dp[w * np_:w * np_ + n]
        dinv = lax.rsqrt(deg)
        dinv_ref[...] = dinv
        hmat_t = lax.dot_general(
            w_ref[...], x_ref[...], (((0,), (1,)), ((), ())),
            preferred_element_type=jnp.float32)
        for f in range(h):
            hp_ref[pl.ds(f * n, n)] = hmat_t[f] * dinv
    return _tc_call(body, (
        jax.ShapeDtypeStruct((n,), jnp.float32),
        jax.ShapeDtypeStruct((h * n,), jnp.float32),
    ))(degp, w1, x)


def _tc_layer1_combine(aggp, hp, dinv, w2, b1, n, np_, h):
    def body(aggp_ref, hp_ref, dinv_ref, w2_ref, b1_ref, gp_ref):
        ap = aggp_ref[...]
        dv = dinv_ref[...]
        w2v = w2_ref[...]
        b1v = b1_ref[...]
        hpv = hp_ref[...]
        g = jnp.zeros((n,), jnp.float32)
        for f in range(h):
            fg, c = f // FG, f % FG
            agg_f = ap[(fg * FG + c) * np_:(fg * FG + c) * np_ + n]
            for es in range(1, ES):
                o = ((es * FG + fg) * FG + c) * np_
                agg_f = agg_f + ap[o:o + n]
            a1_f = jnp.maximum(
                dv * (agg_f + hpv[f * n:f * n + n]) + b1v[f], 0.0)
            g = g + a1_f * w2v[f, 0]
        gp_ref[...] = dv * g
    return _tc_call(body, jax.ShapeDtypeStruct((n,), jnp.float32))(
        aggp, hp, dinv, w2, b1)


def _tc_layer2_combine(agg2p, gp, dinv, b2, n, np_):
    def body(a2_ref, gp_ref, dinv_ref, b2_ref, o_ref):
        ap = a2_ref[...]
        agg2 = gp_ref[...]
        for w in range(NW):
            agg2 = agg2 + ap[w * np_:w * np_ + n]
        o_ref[...] = dinv_ref[...] * agg2 + b2_ref[...][0]
    return _tc_call(body, jax.ShapeDtypeStruct((n,), jnp.float32))(
        agg2p, gp, dinv, b2)


def kernel(x, edge_index, edge_weight, W1, b1, W2, b2):
    n, d = x.shape
    h = W1.shape[1]
    e = edge_index.shape[1]
    assert h == FG * FG
    np_ = ((n + 127) // 128) * 128

    ew = edge_weight.astype(jnp.float32)

    ch = 4000
    while e % ch != 0 or (e // ch) % ES != 0 or ((e // ch) // ES) % 2 != 0:
        ch -= 8

    src, dst = _tc_split_edges(edge_index.astype(jnp.int32), e)

    degp = _build_sc_scalar_agg(e, n, np_, gather=False)(dst, ew)

    dinv, hp = _tc_norm(degp, W1, x, n, np_, h)

    aggp = _build_sc_row_agg(e, n, np_, ch, h)(src, dst, ew, hp)

    gp = _tc_layer1_combine(aggp, hp, dinv, W2, b1, n, np_, h)

    agg2p = _build_sc_scalar_agg(e, n, np_, gather=True)(src, dst, ew, gp)

    return _tc_layer2_combine(agg2p, gp, dinv, b2, n, np_)
